# Initial kernel scaffold; baseline (speedup 1.0000x reference)
import jax
import jax.numpy as jnp
from jax.experimental import pallas as pl


def kernel(inputs, state, gate_kernel_users, gate_kernel_items, gate_bias, candidate_kernel_users, candidate_kernel_items, candidate_bias):
    raise NotImplementedError("write your pallas kernel here")


# speedup vs baseline: 1.6747x; 1.6747x over previous
import functools

import jax
import jax.numpy as jnp
from jax import lax
from jax.experimental import pallas as pl
from jax.experimental.pallas import tpu as pltpu
from jax.experimental.pallas import tpu_sc as plsc

NC = 2
NS = 16
NW = NC * NS


def _sigmoid(x):
    e = jnp.exp(-jnp.abs(x))
    return jnp.where(x >= 0, 1.0 / (1.0 + e), e / (1.0 + e))


def _tanh(x):
    e = jnp.exp(-2.0 * jnp.abs(x))
    t = (1.0 - e) / (1.0 + e)
    return jnp.where(x >= 0, t, -t)


@functools.partial(jax.jit, static_argnums=())
def kernel(inputs, state, gate_kernel_users, gate_kernel_items, gate_bias,
           candidate_kernel_users, candidate_kernel_items, candidate_bias):
    B, H = state.shape
    BPW = B // NW
    u_idx = inputs[:, 0].astype(jnp.int32)
    i_idx = inputs[:, 1].astype(jnp.int32)
    U1 = gate_kernel_users.shape[0]
    I1 = gate_kernel_items.shape[0]
    gku2 = gate_kernel_users.reshape(U1, H * 2 * H)
    cku2 = candidate_kernel_users.reshape(U1, H * H)

    mesh = plsc.VectorSubcoreMesh(
        core_axis_name="c", subcore_axis_name="s",
        num_cores=NC, num_subcores=NS)

    @functools.partial(
        pl.kernel,
        out_type=jax.ShapeDtypeStruct((B, H), jnp.float32),
        mesh=mesh,
        scratch_types=[
            pltpu.VMEM((BPW,), jnp.int32),
            pltpu.VMEM((BPW,), jnp.int32),
            pltpu.VMEM((BPW, H), jnp.float32),
            pltpu.VMEM((BPW, H * 2 * H), jnp.float32),
            pltpu.VMEM((BPW, 2 * H), jnp.float32),
            pltpu.VMEM((BPW, H * H), jnp.float32),
            pltpu.VMEM((BPW, H), jnp.float32),
            pltpu.VMEM((2 * H,), jnp.float32),
            pltpu.VMEM((H,), jnp.float32),
            pltpu.VMEM((BPW, H), jnp.float32),
            pltpu.SemaphoreType.DMA,
            pltpu.SemaphoreType.DMA,
            pltpu.SemaphoreType.DMA,
            pltpu.SemaphoreType.DMA,
        ],
        compiler_params=pltpu.CompilerParams(use_tc_tiling_on_sc=False),
    )
    def run(u_hbm, i_hbm, s_hbm, gku_hbm, gki_hbm, gb_hbm, cku_hbm, cki_hbm,
            cb_hbm, out_hbm, u_v, i_v, s_v, wg_v, gi_v, wc_v, ci_v, gb_v,
            cb_v, o_v, sem0, sem1, sem2, sem3):
        wid = lax.axis_index("s") * NC + lax.axis_index("c")
        base = wid * BPW
        pltpu.sync_copy(u_hbm.at[pl.ds(base, BPW)], u_v)
        pltpu.sync_copy(i_hbm.at[pl.ds(base, BPW)], i_v)
        pltpu.sync_copy(s_hbm.at[pl.ds(base, BPW)], s_v)
        pltpu.sync_copy(gb_hbm, gb_v)
        pltpu.sync_copy(cb_hbm, cb_v)
        cp0 = pltpu.async_copy(gku_hbm.at[u_v], wg_v, sem0)
        cp1 = pltpu.async_copy(cku_hbm.at[u_v], wc_v, sem1)
        cp2 = pltpu.async_copy(gki_hbm.at[i_v], gi_v, sem2)
        cp3 = pltpu.async_copy(cki_hbm.at[i_v], ci_v, sem3)
        cp0.wait()
        cp1.wait()
        cp2.wait()
        cp3.wait()

        gbias_hi = gb_v[pl.ds(H, H)]
        cbias = cb_v[...]

        def elem(b, carry):
            acc_u = gbias_hi + gi_v[b, pl.ds(H, H)]
            acc_c = cbias + ci_v[b]
            sb = s_v[b]
            for h in range(H):
                sh = sb[h]
                acc_u = acc_u + sh * wg_v[b, pl.ds(h * 2 * H + H, H)]
                acc_c = acc_c + sh * wc_v[b, pl.ds(h * H, H)]
            u_gate = _sigmoid(acc_u)
            c = _tanh(acc_c)
            o_v[b] = u_gate * sb + (1.0 - u_gate) * c
            return carry

        lax.fori_loop(0, BPW, elem, 0)
        pltpu.sync_copy(o_v, out_hbm.at[pl.ds(base, BPW)])

    return run(u_idx, i_idx, state, gku2, gate_kernel_items,
               gate_bias, cku2, candidate_kernel_items,
               candidate_bias)


# speedup vs baseline: 2.8323x; 1.6912x over previous
import functools

import jax
import jax.numpy as jnp
from jax import lax
from jax.experimental import pallas as pl
from jax.experimental.pallas import tpu as pltpu
from jax.experimental.pallas import tpu_sc as plsc

NC = 2
NS = 16
NW = NC * NS


def _sigmoid(x):
    e = jnp.exp(-jnp.abs(x))
    return jnp.where(x >= 0, 1.0 / (1.0 + e), e / (1.0 + e))


def _tanh(x):
    e = jnp.exp(-2.0 * jnp.abs(x))
    t = (1.0 - e) / (1.0 + e)
    return jnp.where(x >= 0, t, -t)


@jax.jit
def kernel(inputs, state, gate_kernel_users, gate_kernel_items, gate_bias,
           candidate_kernel_users, candidate_kernel_items, candidate_bias):
    B, H = state.shape
    BPW = B // NW
    u_idx = inputs[:, 0].astype(jnp.int32)
    i_idx = inputs[:, 1].astype(jnp.int32)
    U1 = gate_kernel_users.shape[0]
    gku2 = gate_kernel_users.reshape(U1, H * 2 * H)
    cku2 = candidate_kernel_users.reshape(U1, H * H)
    gi = jnp.take(gate_kernel_items[:, H:], i_idx, axis=0) + gate_bias[H:]
    ci = jnp.take(candidate_kernel_items, i_idx, axis=0) + candidate_bias
    gi1 = gi.reshape(B * H)
    ci1 = ci.reshape(B * H)
    s1 = state.reshape(B * H)

    mesh = plsc.VectorSubcoreMesh(
        core_axis_name="c", subcore_axis_name="s",
        num_cores=NC, num_subcores=NS)

    @functools.partial(
        pl.kernel,
        out_type=jax.ShapeDtypeStruct((B * H,), jnp.float32),
        mesh=mesh,
        scratch_types=[
            pltpu.VMEM((BPW,), jnp.int32),
            pltpu.VMEM((BPW * H,), jnp.float32),
            pltpu.VMEM((BPW, H * 2 * H), jnp.float32),
            pltpu.VMEM((BPW * H,), jnp.float32),
            pltpu.VMEM((BPW, H * H), jnp.float32),
            pltpu.VMEM((BPW * H,), jnp.float32),
            pltpu.VMEM((BPW * H,), jnp.float32),
            pltpu.SemaphoreType.DMA,
            pltpu.SemaphoreType.DMA,
        ],
    )
    def run(u_hbm, s_hbm, gku_hbm, gi_hbm, cku_hbm, ci_hbm, out_hbm,
            u_v, s_v, wg_v, gi_v, wc_v, ci_v, o_v, sem0, sem1):
        wid = lax.axis_index("s") * NC + lax.axis_index("c")
        base = wid * BPW
        pltpu.sync_copy(u_hbm.at[pl.ds(base, BPW)], u_v)
        cp0 = pltpu.async_copy(gku_hbm.at[u_v], wg_v, sem0)
        cp1 = pltpu.async_copy(cku_hbm.at[u_v], wc_v, sem1)
        pltpu.sync_copy(s_hbm.at[pl.ds(base * H, BPW * H)], s_v)
        pltpu.sync_copy(gi_hbm.at[pl.ds(base * H, BPW * H)], gi_v)
        pltpu.sync_copy(ci_hbm.at[pl.ds(base * H, BPW * H)], ci_v)
        cp0.wait()
        cp1.wait()

        def elem(b, carry):
            acc_u = gi_v[pl.ds(b * H, H)]
            acc_c = ci_v[pl.ds(b * H, H)]
            sb = s_v[pl.ds(b * H, H)]
            for h in range(H):
                sh = sb[h]
                acc_u = acc_u + sh * wg_v[b, pl.ds(h * 2 * H + H, H)]
                acc_c = acc_c + sh * wc_v[b, pl.ds(h * H, H)]
            u_gate = _sigmoid(acc_u)
            c = _tanh(acc_c)
            o_v[pl.ds(b * H, H)] = u_gate * sb + (1.0 - u_gate) * c
            return carry

        lax.fori_loop(0, BPW, elem, 0)
        pltpu.sync_copy(o_v, out_hbm.at[pl.ds(base * H, BPW * H)])

    out = run(u_idx, s1, gku2, gi1, cku2, ci1)
    return out.reshape(B, H)
